# Initial kernel scaffold; baseline (speedup 1.0000x reference)
import functools

import jax
import jax.numpy as jnp
from jax import lax
from jax.experimental import pallas as pl
from jax.experimental.pallas import tpu as pltpu
from jax.experimental.pallas import tpu_sc as plsc

_NC = 2
_NS = 16
_LANES = 16
_CH = 80


def _fill_2d(ref, rows, value):
  cols = ref.shape[1]
  vec = jnp.full((_LANES,), value, ref.dtype)

  def body(i, _):
    r = i // (cols // _LANES)
    jcol = (i % (cols // _LANES)) * _LANES
    ref[r, pl.ds(jcol, _LANES)] = vec
    return 0

  lax.fori_loop(0, rows * (cols // _LANES), body, 0)


def _sc_segment_sum(n_nodes, n_edges, width, count_mode):
  half = n_nodes // 2
  per_tile = -(-(half + 8) // _NS)
  per_tile = -(-per_tile // 8) * 8
  acc_rows = per_tile * _NS
  ept = n_edges // _NS
  nch = ept // _CH
  assert ept % _CH == 0 and n_nodes % 2 == 0 and acc_rows > half

  zrows = 64
  mesh = plsc.VectorSubcoreMesh(core_axis_name="c", subcore_axis_name="s")

  @functools.partial(
      pl.kernel,
      out_type=jax.ShapeDtypeStruct((_NC * acc_rows, width), jnp.float32),
      mesh=mesh,
      scratch_types=[
          pltpu.VMEM((_CH,), jnp.int32),
          pltpu.VMEM((_CH,), jnp.int32),
          pltpu.VMEM((_CH,), jnp.int32),
          pltpu.VMEM((_CH, width), jnp.float32),
          pltpu.VMEM((zrows, width), jnp.float32),
          pltpu.VMEM_SHARED((acc_rows, width), jnp.float32),
          pltpu.SemaphoreType.DMA,
      ],
  )
  def seg_sum(xp_hbm, src_hbm, dst_hbm, out_hbm,
              src_v, dstr_v, dst_v, rows_v, zeros_v, acc_sh, sem):
    c = lax.axis_index("c")
    s = lax.axis_index("s")

    if count_mode:
      _fill_2d(rows_v, _CH, 1.0)
    _fill_2d(zeros_v, zrows, 0.0)

    def zero_body(i, _):
      pltpu.sync_copy(zeros_v, acc_sh.at[pl.ds(s * per_tile + i * zrows, zrows)])
      return 0
    lax.fori_loop(0, per_tile // zrows, zero_body, 0)
    rem = per_tile % zrows
    if rem:
      pltpu.sync_copy(zeros_v.at[pl.ds(0, rem)],
                      acc_sh.at[pl.ds(s * per_tile + (per_tile // zrows) * zrows, rem)])
    plsc.subcore_barrier()

    lo = c * half

    def chunk(i, _):
      base = s * ept + i * _CH
      pltpu.sync_copy(dst_hbm.at[pl.ds(base, _CH)], dstr_v)
      if not count_mode:
        pltpu.sync_copy(src_hbm.at[pl.ds(base, _CH)], src_v)

      def vmap_body(j, _):
        d = dstr_v[pl.ds(j * _LANES, _LANES)] - lo
        ok = (d >= 0) & (d < half)
        dst_v[pl.ds(j * _LANES, _LANES)] = jnp.where(ok, d, half)
        return 0
      lax.fori_loop(0, _CH // _LANES, vmap_body, 0)

      if not count_mode:
        pltpu.async_copy(xp_hbm.at[src_v], rows_v, sem).wait()
      pltpu.sync_copy(rows_v, acc_sh.at[dst_v], add=True)
      return 0

    lax.fori_loop(0, nch, chunk, 0)
    plsc.subcore_barrier()

    pltpu.sync_copy(acc_sh.at[pl.ds(s * per_tile, per_tile)],
                    out_hbm.at[pl.ds(c * acc_rows + s * per_tile, per_tile)])

  return seg_sum, acc_rows


@functools.cache
def _build(n_nodes, n_edges, d_model):
  half = n_nodes // 2
  seg_sum, acc_rows = _sc_segment_sum(n_nodes, n_edges, d_model, False)
  cnt_sum, _ = _sc_segment_sum(n_nodes, n_edges, _LANES, True)

  blk = 1000
  grid = (n_nodes // blk,)
  f32 = jnp.float32

  def proj_body(h_ref, w_ref, b_ref, o_ref):
    o_ref[...] = jnp.maximum(
        jnp.dot(h_ref[...], w_ref[...], preferred_element_type=f32)
        + b_ref[...], 0.0)

  proj = pl.pallas_call(
      proj_body,
      grid=grid,
      in_specs=[
          pl.BlockSpec((blk, d_model), lambda i: (i, 0)),
          pl.BlockSpec((d_model, d_model), lambda i: (0, 0)),
          pl.BlockSpec((1, d_model), lambda i: (0, 0)),
      ],
      out_specs=pl.BlockSpec((blk, d_model), lambda i: (i, 0)),
      out_shape=jax.ShapeDtypeStruct((n_nodes, d_model), f32),
  )

  def combine_body(do_ln, s_ref, c_ref, h_ref, wl_ref, wr_ref, bl_ref,
                   g_ref, be_ref, o_ref):
    inv = 1.0 / jnp.maximum(c_ref[...], 1.0)
    agg = s_ref[...] * inv
    t = (jnp.dot(agg, wl_ref[...], preferred_element_type=f32)
         + jnp.dot(h_ref[...], wr_ref[...], preferred_element_type=f32)
         + bl_ref[...])
    if do_ln:
      t = jnp.maximum(t, 0.0)
      mu = jnp.mean(t, axis=-1, keepdims=True)
      var = jnp.mean((t - mu) ** 2, axis=-1, keepdims=True)
      t = (t - mu) * lax.rsqrt(var + 1e-5) * g_ref[...] + be_ref[...]
    o_ref[...] = t

  def make_combine(do_ln):
    return pl.pallas_call(
        functools.partial(combine_body, do_ln),
        grid=grid,
        in_specs=[
            pl.BlockSpec((blk, d_model), lambda i: (i, 0)),
            pl.BlockSpec((blk, 1), lambda i: (i, 0)),
            pl.BlockSpec((blk, d_model), lambda i: (i, 0)),
            pl.BlockSpec((d_model, d_model), lambda i: (0, 0)),
            pl.BlockSpec((d_model, d_model), lambda i: (0, 0)),
            pl.BlockSpec((1, d_model), lambda i: (0, 0)),
            pl.BlockSpec((1, d_model), lambda i: (0, 0)),
            pl.BlockSpec((1, d_model), lambda i: (0, 0)),
        ],
        out_specs=pl.BlockSpec((blk, d_model), lambda i: (i, 0)),
        out_shape=jax.ShapeDtypeStruct((n_nodes, d_model), f32),
    )

  combine_ln = make_combine(True)
  combine_last = make_combine(False)

  def run(x, src, dst, params, n_layers):
    cnt_full = cnt_sum(x[:1], src, dst)
    cnt = jnp.concatenate(
        [cnt_full[:half, :1], cnt_full[acc_rows:acc_rows + half, :1]], axis=0)
    h = x
    for i in range(n_layers):
      xp = proj(h, params['Wp%d' % i], params['bp%d' % i].reshape(1, -1))
      summed_full = seg_sum(xp, src, dst)
      summed = jnp.concatenate(
          [summed_full[:half], summed_full[acc_rows:acc_rows + half]], axis=0)
      if i < n_layers - 1:
        h = combine_ln(summed, cnt, h,
                       params['Wl%d' % i], params['Wr%d' % i],
                       params['bl%d' % i].reshape(1, -1),
                       params['g%d' % i].reshape(1, -1),
                       params['b%d' % i].reshape(1, -1))
      else:
        zero = jnp.zeros((1, h.shape[1]), f32)
        h = combine_last(summed, cnt, h,
                         params['Wl%d' % i], params['Wr%d' % i],
                         params['bl%d' % i].reshape(1, -1), zero, zero)
    return h

  return run


def kernel(x, edge_index, params):
  n_nodes, d_model = x.shape
  n_edges = edge_index.shape[1]
  n_layers = len([k for k in params if k.startswith('Wp')])
  run = _build(n_nodes, n_edges, d_model)
  return run(x, edge_index[0], edge_index[1], params, n_layers)


# speedup vs baseline: 2.0934x; 2.0934x over previous
import functools

import jax
import jax.numpy as jnp
from jax import lax
from jax.experimental import pallas as pl
from jax.experimental.pallas import tpu as pltpu
from jax.experimental.pallas import tpu_sc as plsc

_NC = 2
_NS = 16
_NW = _NC * _NS
_LANES = 16
_CH = 128
_CAP = 8192
_ECH = 640


def _mesh():
  return plsc.VectorSubcoreMesh(core_axis_name="c", subcore_axis_name="s",
                                num_cores=_NC, num_subcores=_NS)


def _worker():
  return lax.axis_index("c") * _NS + lax.axis_index("s")


def _sc_route(n_nodes, n_edges, pt):
  assert n_edges % _ECH == 0

  @functools.partial(
      pl.kernel,
      out_type=(jax.ShapeDtypeStruct((_NW * _CAP,), jnp.int32),
                jax.ShapeDtypeStruct((_NW * _CAP,), jnp.int32),
                jax.ShapeDtypeStruct((_NW * 128,), jnp.int32)),
      mesh=_mesh(),
      compiler_params=pltpu.CompilerParams(needs_layout_passes=False),
      scratch_types=[
          pltpu.VMEM((_ECH,), jnp.int32),
          pltpu.VMEM((_ECH,), jnp.int32),
          pltpu.VMEM((_CAP,), jnp.int32),
          pltpu.VMEM((_CAP,), jnp.int32),
          pltpu.VMEM((16,), jnp.int32),
      ],
  )
  def route(src_hbm, dst_hbm, osrc, odst, onch, db, sb, wsrc, wdst, nv):
    w = _worker()
    lo = w * pt

    def fill(i, _):
      wdst[pl.ds(i * 16, 16)] = jnp.full((16,), pt, jnp.int32)
      wsrc[pl.ds(i * 16, 16)] = jnp.full((16,), lo, jnp.int32)
      return 0
    lax.fori_loop(0, _CAP // 16, fill, 0)

    def chunk(i, wp):
      pltpu.sync_copy(dst_hbm.at[pl.ds(i * _ECH, _ECH)], db)
      pltpu.sync_copy(src_hbm.at[pl.ds(i * _ECH, _ECH)], sb)

      def sub(j, wp):
        d = db[pl.ds(j * 16, 16)] - lo
        sv = sb[pl.ds(j * 16, 16)]
        m = (d >= 0) & (d < pt)
        plsc.store_compressed(wdst.at[pl.ds(wp, 16)], d, mask=m)
        plsc.store_compressed(wsrc.at[pl.ds(wp, 16)], sv, mask=m)
        npop = plsc.all_reduce_population_count(m)
        return jnp.minimum(wp + jnp.max(npop), _CAP - 16)
      return lax.fori_loop(0, _ECH // 16, sub, wp)

    wp = lax.fori_loop(0, n_edges // _ECH, chunk, 0)
    nch = (wp + _CH - 1) // _CH
    pltpu.sync_copy(wsrc, osrc.at[pl.ds(w * _CAP, _CAP)])
    pltpu.sync_copy(wdst, odst.at[pl.ds(w * _CAP, _CAP)])
    nv[...] = jnp.full((16,), nch, jnp.int32)
    pltpu.sync_copy(nv, onch.at[pl.ds(w * 128, 16)])

  return route


def _sc_aggregate(n_nodes, d_model, pt, with_counts):
  padn = _NW * pt
  acc_rows = pt + 8

  out_type = [jax.ShapeDtypeStruct((padn, d_model), jnp.float32)]
  scratch = [
      pltpu.VMEM((_CH,), jnp.int32),
      pltpu.VMEM((_CH,), jnp.int32),
      pltpu.VMEM((_CH, d_model), jnp.float32),
      pltpu.VMEM((acc_rows, d_model), jnp.float32),
      pltpu.VMEM((16,), jnp.int32),
      pltpu.SemaphoreType.DMA,
  ]
  if with_counts:
    out_type.append(jax.ShapeDtypeStruct((padn * _LANES,), jnp.float32))
    scratch.append(pltpu.VMEM((acc_rows * _LANES,), jnp.float32))

  @functools.partial(
      pl.kernel,
      out_type=tuple(out_type) if with_counts else out_type[0],
      mesh=_mesh(),
      compiler_params=pltpu.CompilerParams(needs_layout_passes=False),
      scratch_types=scratch,
  )
  def acc_kernel(xp_hbm, wsrc_hbm, wdst_hbm, nch_hbm, *rest):
    if with_counts:
      out_hbm, cnt_hbm, sidx, ldst, rows, acc, nv, sem, acc1 = rest
    else:
      out_hbm, sidx, ldst, rows, acc, nv, sem = rest
    w = _worker()

    zero16 = jnp.zeros((16,), jnp.float32)
    ones16 = jnp.ones((16,), jnp.float32)

    def zf(i, _):
      r = i // (d_model // 16)
      col = (i % (d_model // 16)) * 16
      acc[r, pl.ds(col, 16)] = zero16
      return 0
    lax.fori_loop(0, acc_rows * (d_model // 16), zf, 0)
    if with_counts:
      def zf1(i, _):
        acc1[pl.ds(i * 16, 16)] = zero16
        return 0
      lax.fori_loop(0, acc_rows, zf1, 0)

    pltpu.sync_copy(nch_hbm.at[pl.ds(w * 128, 16)], nv)
    nch = jnp.max(nv[...])

    def chunk(i, _):
      pltpu.sync_copy(wsrc_hbm.at[pl.ds(w * _CAP + i * _CH, _CH)], sidx)
      pltpu.sync_copy(wdst_hbm.at[pl.ds(w * _CAP + i * _CH, _CH)], ldst)
      pltpu.async_copy(xp_hbm.at[sidx], rows, sem).wait()

      def group(g, _):
        r16 = ldst[pl.ds(g * 16, 16)]
        for l in range(16):
          e = g * 16 + l
          r = jnp.max(jnp.where(lax.iota(jnp.int32, 16) == l, r16, 0))
          for j in range(d_model // 16):
            plsc.addupdate(acc.at[r, pl.ds(j * 16, 16)],
                           rows[e, pl.ds(j * 16, 16)])
          if with_counts:
            plsc.addupdate(acc1.at[pl.ds(r * 16, 16)], ones16)
        return 0
      lax.fori_loop(0, _CH // 16, group, 0)
      return 0

    lax.fori_loop(0, nch, chunk, 0)
    pltpu.sync_copy(acc.at[pl.ds(0, pt)], out_hbm.at[pl.ds(w * pt, pt)])
    if with_counts:
      pltpu.sync_copy(acc1.at[pl.ds(0, pt * _LANES)],
                      cnt_hbm.at[pl.ds(w * pt * _LANES, pt * _LANES)])

  return acc_kernel, padn


@functools.cache
def _build(n_nodes, n_edges, d_model):
  pt = -(-n_nodes // (_NW * 8)) * 8
  route = _sc_route(n_nodes, n_edges, pt)
  agg0, padn = _sc_aggregate(n_nodes, d_model, pt, True)
  agg, _ = _sc_aggregate(n_nodes, d_model, pt, False)

  blk = max(b for b in range(8, 2049, 8) if n_nodes % b == 0)
  grid = (n_nodes // blk,)
  f32 = jnp.float32

  def proj_body(h_ref, w_ref, b_ref, o_ref):
    o_ref[...] = jnp.maximum(
        jnp.dot(h_ref[...], w_ref[...], preferred_element_type=f32)
        + b_ref[...], 0.0)

  proj = pl.pallas_call(
      proj_body,
      grid=grid,
      in_specs=[
          pl.BlockSpec((blk, d_model), lambda i: (i, 0)),
          pl.BlockSpec((d_model, d_model), lambda i: (0, 0)),
          pl.BlockSpec((1, d_model), lambda i: (0, 0)),
      ],
      out_specs=pl.BlockSpec((blk, d_model), lambda i: (i, 0)),
      out_shape=jax.ShapeDtypeStruct((n_nodes, d_model), f32),
  )

  def cnt_body(c_ref, o_ref):
    o_ref[...] = c_ref[:, :1]

  cnt_reduce = pl.pallas_call(
      cnt_body,
      grid=grid,
      in_specs=[pl.BlockSpec((blk, _LANES), lambda i: (i, 0))],
      out_specs=pl.BlockSpec((blk, 1), lambda i: (i, 0)),
      out_shape=jax.ShapeDtypeStruct((n_nodes, 1), f32),
  )

  def combine_body(do_ln, s_ref, c_ref, h_ref, wl_ref, wr_ref, bl_ref,
                   g_ref, be_ref, o_ref):
    inv = 1.0 / jnp.maximum(c_ref[...], 1.0)
    agg_blk = s_ref[...] * inv
    t = (jnp.dot(agg_blk, wl_ref[...], preferred_element_type=f32)
         + jnp.dot(h_ref[...], wr_ref[...], preferred_element_type=f32)
         + bl_ref[...])
    if do_ln:
      t = jnp.maximum(t, 0.0)
      mu = jnp.mean(t, axis=-1, keepdims=True)
      var = jnp.mean((t - mu) ** 2, axis=-1, keepdims=True)
      t = (t - mu) * lax.rsqrt(var + 1e-5) * g_ref[...] + be_ref[...]
    o_ref[...] = t

  def make_combine(do_ln):
    return pl.pallas_call(
        functools.partial(combine_body, do_ln),
        grid=grid,
        in_specs=[
            pl.BlockSpec((blk, d_model), lambda i: (i, 0)),
            pl.BlockSpec((blk, 1), lambda i: (i, 0)),
            pl.BlockSpec((blk, d_model), lambda i: (i, 0)),
            pl.BlockSpec((d_model, d_model), lambda i: (0, 0)),
            pl.BlockSpec((d_model, d_model), lambda i: (0, 0)),
            pl.BlockSpec((1, d_model), lambda i: (0, 0)),
            pl.BlockSpec((1, d_model), lambda i: (0, 0)),
            pl.BlockSpec((1, d_model), lambda i: (0, 0)),
        ],
        out_specs=pl.BlockSpec((blk, d_model), lambda i: (i, 0)),
        out_shape=jax.ShapeDtypeStruct((n_nodes, d_model), f32),
    )

  combine_ln = make_combine(True)
  combine_last = make_combine(False)

  def run(x, src, dst, params, n_layers):
    wl_src, wl_dst, nchs = route(src, dst)
    cnt = None
    h = x
    for i in range(n_layers):
      xp = proj(h, params['Wp%d' % i], params['bp%d' % i].reshape(1, -1))
      if i == 0:
        summed, cnt16 = agg0(xp, wl_src, wl_dst, nchs)
        cnt = cnt_reduce(cnt16.reshape(-1, _LANES))
      else:
        summed = agg(xp, wl_src, wl_dst, nchs)
      if i < n_layers - 1:
        h = combine_ln(summed, cnt, h,
                       params['Wl%d' % i], params['Wr%d' % i],
                       params['bl%d' % i].reshape(1, -1),
                       params['g%d' % i].reshape(1, -1),
                       params['b%d' % i].reshape(1, -1))
      else:
        zero = jnp.zeros((1, h.shape[1]), f32)
        h = combine_last(summed, cnt, h,
                         params['Wl%d' % i], params['Wr%d' % i],
                         params['bl%d' % i].reshape(1, -1), zero, zero)
    return h

  return run


def kernel(x, edge_index, params):
  n_nodes, d_model = x.shape
  n_edges = edge_index.shape[1]
  n_layers = len([k for k in params if k.startswith('Wp')])
  run = _build(n_nodes, n_edges, d_model)
  return run(x, edge_index[0], edge_index[1], params, n_layers)


# speedup vs baseline: 2.8227x; 1.3484x over previous
import functools

import jax
import jax.numpy as jnp
from jax import lax
from jax.experimental import pallas as pl
from jax.experimental.pallas import tpu as pltpu
from jax.experimental.pallas import tpu_sc as plsc

_NC = 2
_NS = 16
_NW = _NC * _NS
_NR = 2 * _NW
_LANES = 16
_CH = 128
_CAP = 6400
_ECH = 1280


def _mesh():
  return plsc.VectorSubcoreMesh(core_axis_name="c", subcore_axis_name="s",
                                num_cores=_NC, num_subcores=_NS)


def _worker():
  return lax.axis_index("c") * _NS + lax.axis_index("s")


def _sc_route(n_nodes, n_edges, pr):
  nech = n_edges // _ECH
  assert n_edges % _ECH == 0 and nech % 2 == 1 and _ECH % 128 == 0

  @functools.partial(
      pl.kernel,
      out_type=(jax.ShapeDtypeStruct((_NR * _CAP,), jnp.int32),
                jax.ShapeDtypeStruct((_NR * _CAP,), jnp.int32),
                jax.ShapeDtypeStruct((_NR * 128,), jnp.int32)),
      mesh=_mesh(),
      compiler_params=pltpu.CompilerParams(needs_layout_passes=False),
      scratch_types=[
          pltpu.VMEM((_ECH,), jnp.int32),
          pltpu.VMEM((_ECH,), jnp.int32),
          pltpu.VMEM((_ECH,), jnp.int32),
          pltpu.VMEM((_ECH,), jnp.int32),
          pltpu.VMEM((_CAP,), jnp.int32),
          pltpu.VMEM((_CAP,), jnp.int32),
          pltpu.VMEM((_CAP,), jnp.int32),
          pltpu.VMEM((_CAP,), jnp.int32),
          pltpu.VMEM((16,), jnp.int32),
          pltpu.SemaphoreType.DMA,
          pltpu.SemaphoreType.DMA,
      ],
  )
  def route(src_hbm, dst_hbm, osrc, odst, onch,
            db0, db1, sb0, sb1, wsa, wda, wsb, wdb, nv, sem0, sem1):
    w = _worker()
    lo = w * 2 * pr
    sems = (sem0, sem1)

    def fill(i, _):
      dummy = jnp.full((16,), pr, jnp.int32)
      wda[pl.ds(i * 16, 16)] = dummy
      wdb[pl.ds(i * 16, 16)] = dummy
      srow = jnp.full((16,), lo, jnp.int32)
      wsa[pl.ds(i * 16, 16)] = srow
      wsb[pl.ds(i * 16, 16)] = srow
      return 0
    lax.fori_loop(0, _CAP // 16, fill, 0)

    dbs = (db0, db1)
    sbs = (sb0, sb1)

    def load(i, b):
      pltpu.async_copy(dst_hbm.at[pl.ds(i * _ECH, _ECH)], dbs[b], sems[b])
      pltpu.async_copy(src_hbm.at[pl.ds(i * _ECH, _ECH)], sbs[b], sems[b])

    def wait(i, b):
      pltpu.make_async_copy(dst_hbm.at[pl.ds(i * _ECH, _ECH)], dbs[b],
                            sems[b]).wait()
      pltpu.make_async_copy(src_hbm.at[pl.ds(i * _ECH, _ECH)], sbs[b],
                            sems[b]).wait()

    def scan(b, wps):
      def sub(j, wps):
        wpa, wpb = wps
        d = dbs[b][pl.ds(j * 16, 16)] - lo
        sv = sbs[b][pl.ds(j * 16, 16)]
        ma = (d >= 0) & (d < pr)
        mb = (d >= pr) & (d < 2 * pr)
        plsc.store_compressed(wda.at[pl.ds(wpa, 16)], d, mask=ma)
        plsc.store_compressed(wsa.at[pl.ds(wpa, 16)], sv, mask=ma)
        plsc.store_compressed(wdb.at[pl.ds(wpb, 16)], d - pr, mask=mb)
        plsc.store_compressed(wsb.at[pl.ds(wpb, 16)], sv, mask=mb)
        na = plsc.all_reduce_population_count(ma)[0]
        nb = plsc.all_reduce_population_count(mb)[0]
        wpa = jnp.minimum(wpa + na, _CAP - 16)
        wpb = jnp.minimum(wpb + nb, _CAP - 16)
        return wpa, wpb
      return lax.fori_loop(0, _ECH // 16, sub, wps)

    load(0, 0)

    def pair(p, wps):
      load(2 * p + 1, 1)
      wait(2 * p, 0)
      wps = scan(0, wps)
      load(2 * p + 2, 0)
      wait(2 * p + 1, 1)
      return scan(1, wps)

    wps = lax.fori_loop(0, (nech - 1) // 2, pair, (0, 0))
    wait(nech - 1, 0)
    wpa, wpb = scan(0, wps)

    def put_nch(rid, wp):
      nch = jnp.clip(((wp + 2 * _CH - 1) // (2 * _CH)) * 2, 2, _CAP // _CH)
      nv[...] = jnp.full((16,), nch, jnp.int32)
      pltpu.sync_copy(nv, onch.at[pl.ds(rid * 128, 16)])

    ra = 2 * w
    rb = 2 * w + 1
    pltpu.sync_copy(wsa, osrc.at[pl.ds(ra * _CAP, _CAP)])
    pltpu.sync_copy(wda, odst.at[pl.ds(ra * _CAP, _CAP)])
    pltpu.sync_copy(wsb, osrc.at[pl.ds(rb * _CAP, _CAP)])
    pltpu.sync_copy(wdb, odst.at[pl.ds(rb * _CAP, _CAP)])
    put_nch(ra, wpa)
    put_nch(rb, wpb)

  return route


def _sc_aggregate(n_nodes, d_model, pr, with_counts):
  padn = _NR * pr
  acc_rows = pr + 8

  out_type = [jax.ShapeDtypeStruct((padn, d_model), jnp.float32)]
  scratch = [
      pltpu.VMEM((_CAP,), jnp.int32),
      pltpu.VMEM((_CAP,), jnp.int32),
      pltpu.VMEM((2, _CH, d_model), jnp.float32),
      pltpu.VMEM((acc_rows, d_model), jnp.float32),
      pltpu.VMEM((16,), jnp.int32),
      pltpu.SemaphoreType.DMA,
      pltpu.SemaphoreType.DMA,
  ]
  if with_counts:
    out_type.append(jax.ShapeDtypeStruct((padn * _LANES,), jnp.float32))
    scratch.append(pltpu.VMEM((acc_rows * _LANES,), jnp.float32))

  @functools.partial(
      pl.kernel,
      out_type=tuple(out_type) if with_counts else out_type[0],
      mesh=_mesh(),
      compiler_params=pltpu.CompilerParams(needs_layout_passes=False),
      scratch_types=scratch,
  )
  def acc_kernel(xp_hbm, wsrc_hbm, wdst_hbm, nch_hbm, *rest):
    if with_counts:
      out_hbm, cnt_hbm, wsl, wdl, rows2, acc, nv, sem0, sem1, acc1 = rest
    else:
      out_hbm, wsl, wdl, rows2, acc, nv, sem0, sem1 = rest
    w = _worker()
    sems = (sem0, sem1)

    zero16 = jnp.zeros((16,), jnp.float32)
    ones16 = jnp.ones((16,), jnp.float32)

    for rr in range(2):
      rid = 2 * w + rr
      base = rid * _CAP

      def zf(i, _):
        r = i // (d_model // 16)
        col = (i % (d_model // 16)) * 16
        acc[r, pl.ds(col, 16)] = zero16
        return 0
      lax.fori_loop(0, acc_rows * (d_model // 16), zf, 0)
      if with_counts:
        def zf1(i, _):
          acc1[pl.ds(i * 16, 16)] = zero16
          return 0
        lax.fori_loop(0, acc_rows, zf1, 0)

      pltpu.sync_copy(wsrc_hbm.at[pl.ds(base, _CAP)], wsl)
      pltpu.sync_copy(wdst_hbm.at[pl.ds(base, _CAP)], wdl)
      pltpu.sync_copy(nch_hbm.at[pl.ds(rid * 128, 16)], nv)
      nch = nv[...][0]

      def start(i, b):
        off = pl.multiple_of(i * _CH, _CH)
        pltpu.async_copy(xp_hbm.at[wsl.at[pl.ds(off, _CH)]],
                         rows2.at[b], sems[b])

      def do_chunk(i, b):
        @pl.when(i + 1 < nch)
        def _():
          start(i + 1, 1 - b)
        off = pl.multiple_of(i * _CH, _CH)
        pltpu.make_async_copy(xp_hbm.at[wsl.at[pl.ds(off, _CH)]],
                              rows2.at[b], sems[b]).wait()
        rows = rows2.at[b]

        def group(g, _):
          r16 = wdl[pl.ds(i * _CH + g * 16, 16)]
          for l in range(16):
            e = g * 16 + l
            r = r16[l]
            for j in range(d_model // 16):
              plsc.addupdate(acc.at[r, pl.ds(j * 16, 16)],
                             rows[e, pl.ds(j * 16, 16)])
            if with_counts:
              plsc.addupdate(acc1.at[pl.ds(r * 16, 16)], ones16)
          return 0
        lax.fori_loop(0, _CH // 16, group, 0)

      start(0, 0)

      def pair(p, _):
        do_chunk(2 * p, 0)
        do_chunk(2 * p + 1, 1)
        return 0
      lax.fori_loop(0, nch // 2, pair, 0)

      pltpu.sync_copy(acc.at[pl.ds(0, pr)],
                      out_hbm.at[pl.ds(rid * pr, pr)])
      if with_counts:
        pltpu.sync_copy(acc1.at[pl.ds(0, pr * _LANES)],
                        cnt_hbm.at[pl.ds(rid * pr * _LANES, pr * _LANES)])

  return acc_kernel, padn


@functools.cache
def _build(n_nodes, n_edges, d_model):
  pr = -(-n_nodes // (_NR * 8)) * 8
  route = _sc_route(n_nodes, n_edges, pr)
  agg0, padn = _sc_aggregate(n_nodes, d_model, pr, True)
  agg, _ = _sc_aggregate(n_nodes, d_model, pr, False)

  blk = max(b for b in range(8, 2049, 8) if n_nodes % b == 0)
  grid = (n_nodes // blk,)
  f32 = jnp.float32

  def proj_body(h_ref, w_ref, b_ref, o_ref):
    o_ref[...] = jnp.maximum(
        jnp.dot(h_ref[...], w_ref[...], preferred_element_type=f32)
        + b_ref[...], 0.0)

  proj = pl.pallas_call(
      proj_body,
      grid=grid,
      in_specs=[
          pl.BlockSpec((blk, d_model), lambda i: (i, 0)),
          pl.BlockSpec((d_model, d_model), lambda i: (0, 0)),
          pl.BlockSpec((1, d_model), lambda i: (0, 0)),
      ],
      out_specs=pl.BlockSpec((blk, d_model), lambda i: (i, 0)),
      out_shape=jax.ShapeDtypeStruct((n_nodes, d_model), f32),
  )

  def cnt_body(c_ref, o_ref):
    o_ref[...] = c_ref[:, :1]

  cnt_reduce = pl.pallas_call(
      cnt_body,
      grid=grid,
      in_specs=[pl.BlockSpec((blk, _LANES), lambda i: (i, 0))],
      out_specs=pl.BlockSpec((blk, 1), lambda i: (i, 0)),
      out_shape=jax.ShapeDtypeStruct((n_nodes, 1), f32),
  )

  def combine_body(do_ln, s_ref, c_ref, h_ref, wl_ref, wr_ref, bl_ref,
                   g_ref, be_ref, o_ref):
    inv = 1.0 / jnp.maximum(c_ref[...], 1.0)
    agg_blk = s_ref[...] * inv
    t = (jnp.dot(agg_blk, wl_ref[...], preferred_element_type=f32)
         + jnp.dot(h_ref[...], wr_ref[...], preferred_element_type=f32)
         + bl_ref[...])
    if do_ln:
      t = jnp.maximum(t, 0.0)
      mu = jnp.mean(t, axis=-1, keepdims=True)
      var = jnp.mean((t - mu) ** 2, axis=-1, keepdims=True)
      t = (t - mu) * lax.rsqrt(var + 1e-5) * g_ref[...] + be_ref[...]
    o_ref[...] = t

  def make_combine(do_ln):
    return pl.pallas_call(
        functools.partial(combine_body, do_ln),
        grid=grid,
        in_specs=[
            pl.BlockSpec((blk, d_model), lambda i: (i, 0)),
            pl.BlockSpec((blk, 1), lambda i: (i, 0)),
            pl.BlockSpec((blk, d_model), lambda i: (i, 0)),
            pl.BlockSpec((d_model, d_model), lambda i: (0, 0)),
            pl.BlockSpec((d_model, d_model), lambda i: (0, 0)),
            pl.BlockSpec((1, d_model), lambda i: (0, 0)),
            pl.BlockSpec((1, d_model), lambda i: (0, 0)),
            pl.BlockSpec((1, d_model), lambda i: (0, 0)),
        ],
        out_specs=pl.BlockSpec((blk, d_model), lambda i: (i, 0)),
        out_shape=jax.ShapeDtypeStruct((n_nodes, d_model), f32),
    )

  combine_ln = make_combine(True)
  combine_last = make_combine(False)

  def run(x, src, dst, params, n_layers):
    wl_src, wl_dst, nchs = route(src, dst)
    cnt = None
    h = x
    for i in range(n_layers):
      xp = proj(h, params['Wp%d' % i], params['bp%d' % i].reshape(1, -1))
      if i == 0:
        summed, cnt16 = agg0(xp, wl_src, wl_dst, nchs)
        cnt = cnt_reduce(cnt16.reshape(-1, _LANES))
      else:
        summed = agg(xp, wl_src, wl_dst, nchs)
      if i < n_layers - 1:
        h = combine_ln(summed, cnt, h,
                       params['Wl%d' % i], params['Wr%d' % i],
                       params['bl%d' % i].reshape(1, -1),
                       params['g%d' % i].reshape(1, -1),
                       params['b%d' % i].reshape(1, -1))
      else:
        zero = jnp.zeros((1, h.shape[1]), f32)
        h = combine_last(summed, cnt, h,
                         params['Wl%d' % i], params['Wr%d' % i],
                         params['bl%d' % i].reshape(1, -1), zero, zero)
    return h

  return run


def kernel(x, edge_index, params):
  n_nodes, d_model = x.shape
  n_edges = edge_index.shape[1]
  n_layers = len([k for k in params if k.startswith('Wp')])
  run = _build(n_nodes, n_edges, d_model)
  return run(x, edge_index[0], edge_index[1], params, n_layers)
